# baseline (speedup 1.0000x reference)
import functools

import jax
import jax.numpy as jnp
from jax import lax
from jax.experimental import pallas as pl
from jax.experimental.pallas import tpu as pltpu
from jax.experimental.pallas import tpu_sc as plsc

_VOCAB = 100000
_DIM = 64
_B = 1024
_L = 20
_VT = 1024
_NV = (_VOCAB + _VT - 1) // _VT
_VPAD = _NV * _VT
_REM = _VOCAB - (_NV - 1) * _VT


def _sc_embed_mean(table, idx_flat):
    nw = 32
    per_w = _B // nw
    mesh = plsc.VectorSubcoreMesh(core_axis_name="c", subcore_axis_name="s")

    @functools.partial(
        pl.kernel,
        out_type=jax.ShapeDtypeStruct((_B, _DIM), jnp.float32),
        mesh=mesh,
        scratch_types=[
            pltpu.VMEM((per_w * _L,), jnp.int32),
            pltpu.VMEM((per_w * _L, _DIM), jnp.float32),
            pltpu.VMEM((per_w, _DIM), jnp.float32),
            pltpu.SemaphoreType.DMA,
        ],
        compiler_params=pltpu.CompilerParams(use_tc_tiling_on_sc=False),
    )
    def k(table_hbm, idx_hbm, out_hbm, idx_v, rows_v, acc_v, sem):
        wid = lax.axis_index("s") * 2 + lax.axis_index("c")
        base = wid * per_w
        pltpu.sync_copy(idx_hbm.at[pl.ds(base * _L, per_w * _L)], idx_v)
        pltpu.async_copy(table_hbm.at[idx_v], rows_v, sem).wait()

        @pl.loop(0, per_w)
        def _(b):
            r0 = b * _L
            for d in range(_DIM // 16):
                sl = pl.ds(d * 16, 16)
                acc = rows_v[r0, sl]
                for l in range(1, _L):
                    acc = acc + rows_v[r0 + l, sl]
                acc_v[b, sl] = acc * (1.0 / _L)

        pltpu.sync_copy(acc_v, out_hbm.at[pl.ds(base, per_w)])

    return k(table, idx_flat)


def _tc_body(emb_ref, w_ref, b_ref, out_ref, bt_scr, ebf_scr, m_scr, s_scr,
             c_scr):
    p = pl.program_id(0)
    v = pl.program_id(1)

    @pl.when((p == 0) & (v == 0))
    def _():
        ebf_scr[...] = emb_ref[...].astype(jnp.bfloat16)
        m_scr[...] = jnp.full((_B, 128), -1e30, jnp.float32)
        s_scr[...] = jnp.zeros((_B, 128), jnp.float32)
        for t in range(_NV - 1):
            bt_scr[t] = b_ref[:, t * _VT:(t + 1) * _VT]
        bt_scr[_NV - 1] = jnp.concatenate(
            [b_ref[:, (_NV - 1) * _VT:],
             jnp.full((1, _VPAD - _VOCAB), -1e30, jnp.float32)], axis=1)

    def logits_tile():
        wc = w_ref[...].astype(jnp.bfloat16)
        row = lax.broadcasted_iota(jnp.int32, (_VT, _DIM), 0)
        wc = jnp.where((v < _NV - 1) | (row < _REM), wc,
                       jnp.zeros_like(wc))
        acc = lax.dot_general(
            ebf_scr[...], wc, (((1,), (1,)), ((), ())),
            preferred_element_type=jnp.float32,
        )
        return acc + bt_scr[v]

    @pl.when(p == 0)
    def _():
        lg = logits_tile()
        m_old = m_scr[:, 0:1]
        s_old = s_scr[:, 0:1]
        mx = jnp.max(lg, axis=1, keepdims=True)
        m_new = jnp.maximum(m_old, mx)
        s_new = s_old * jnp.exp(m_old - m_new) + jnp.sum(
            jnp.exp(lg - m_new), axis=1, keepdims=True)
        m_scr[...] = jnp.broadcast_to(m_new, (_B, 128))
        s_scr[...] = jnp.broadcast_to(s_new, (_B, 128))

        @pl.when(v == _NV - 1)
        def _():
            c_scr[...] = jnp.broadcast_to(m_new + jnp.log(s_new), (_B, 128))

    @pl.when(p == 1)
    def _():
        out_ref[...] = logits_tile() - c_scr[:, 0:1]


def _tc_linear_logsoftmax(embeds, w, bias2d):
    return pl.pallas_call(
        _tc_body,
        grid=(2, _NV),
        in_specs=[
            pl.BlockSpec((_B, _DIM), lambda p, v: (0, 0)),
            pl.BlockSpec((_VT, _DIM), lambda p, v: (v, 0)),
            pl.BlockSpec((1, _VOCAB), lambda p, v: (0, 0)),
        ],
        out_specs=pl.BlockSpec(
            (_B, _VT), lambda p, v: (0, jnp.where(p == 0, 0, v))),
        out_shape=jax.ShapeDtypeStruct((_B, _VOCAB), jnp.float32),
        scratch_shapes=[
            pltpu.VMEM((_NV, 1, _VT), jnp.float32),
            pltpu.VMEM((_B, _DIM), jnp.bfloat16),
            pltpu.VMEM((_B, 128), jnp.float32),
            pltpu.VMEM((_B, 128), jnp.float32),
            pltpu.VMEM((_B, 128), jnp.float32),
        ],
    )(embeds, w, bias2d)


def kernel(input_idx, embedding_weight, linear1_weight, linear1_bias):
    idx_flat = input_idx.reshape(-1).astype(jnp.int32)
    embeds = _sc_embed_mean(embedding_weight, idx_flat)
    return _tc_linear_logsoftmax(
        embeds, linear1_weight, linear1_bias.reshape(1, _VOCAB))
